# baseline (speedup 1.0000x reference)
import jax
import jax.numpy as jnp
from jax import lax
from jax.experimental import pallas as pl
from jax.experimental.pallas import tpu as pltpu
from jax.experimental.pallas import tpu_sc as plsc

NVOCAB = 1000000
NEMBED = 64
NTOKEN = 200
BATCH = 4096

_NUM_WORKERS = 32
_LANES = BATCH // _NUM_WORKERS
_T2ROWS = NVOCAB // 2
_NTB = NVOCAB // 128
_TB_PER_W = _NTB // _NUM_WORKERS
_TB_REM = _NTB % _NUM_WORKERS
_TAIL = NVOCAB - _NTB * 128


def _transpose_kernel(tabT_hbm, tail_hbm, out_hbm, src, dst, tail_v,
                      gsem, osem):
    nc = 2
    wid = lax.axis_index("s") * nc + lax.axis_index("c")

    iota16 = lax.iota(jnp.int32, 16)
    cvecs = [iota16 + (g * 16) for g in range(4)]

    def gdesc(tb, k):
        return pltpu.make_async_copy(
            tabT_hbm.at[:, pl.ds(tb * 128, 128)], src[k], gsem)

    def odesc(tb, k):
        return pltpu.make_async_copy(
            dst[k], out_hbm.at[pl.ds(tb * 64, 64)], osem)

    def transpose(k):
        def r_body(r, c):
            for h in range(2):
                col = 2 * r + h
                for g in range(4):
                    val = plsc.load_gather(src[k], [cvecs[g], iota16 * 0 + col])
                    dst[k][r, pl.ds(h * NEMBED + g * 16, 16)] = val
            return c

        lax.fori_loop(0, NEMBED, r_body, 0, unroll=False)

    n_iters = _TB_PER_W + 1

    def tb_of(i):
        return wid + i * _NUM_WORKERS

    def start(i, k):
        tb = tb_of(i)

        @pl.when(tb < _NTB)
        def _():
            gdesc(tb, k).start()

    def finish(i, k):
        tb = tb_of(i)

        @pl.when(tb < _NTB)
        def _():
            gdesc(tb, k).wait()

            @pl.when(i >= 2)
            def _():
                odesc(tb_of(i - 2), k).wait()

            transpose(k)
            odesc(tb, k).start()

    start(0, 0)
    start(1, 1)

    def body2(i2, c):
        for par in range(2):
            i = i2 * 2 + par
            finish(i, par)
            start(i + 2, par)
        return c

    n_pairs = (n_iters + 1) // 2
    lax.fori_loop(0, n_pairs, body2, 0, unroll=False)

    for i in (2 * n_pairs - 4, 2 * n_pairs - 3, 2 * n_pairs - 2):
        tb = tb_of(i)

        @pl.when((tb < _NTB) & (tb + 2 * _NUM_WORKERS >= _NTB))
        def _():
            odesc(tb, i % 2).wait()

    @pl.when(wid == _NTB % _NUM_WORKERS)
    def _():
        pltpu.sync_copy(tail_hbm, tail_v)

        def r_body(r, c):
            for h in range(2):
                col = 2 * r + h
                for g in range(4):
                    dst[0][r, pl.ds(h * NEMBED + g * 16, 16)] = (
                        tail_v[col, pl.ds(g * 16, 16)])
            return c

        lax.fori_loop(0, _TAIL // 2, r_body, 0, unroll=False)
        pltpu.sync_copy(dst[0].at[pl.ds(0, _TAIL // 2)],
                        out_hbm.at[pl.ds(_NTB * 64, _TAIL // 2)])


def _gather_kernel(tokT_hbm, tab2_hbm, pos_hbm, out_hbm,
                   idx_all, pos_v, gidx, emb, outb, gsem, osem):
    nc = 2
    wid = lax.axis_index("s") * nc + lax.axis_index("c")
    b0 = wid * _LANES

    pltpu.sync_copy(tokT_hbm.at[:, pl.ds(b0, _LANES)], idx_all)
    pltpu.sync_copy(pos_hbm, pos_v)

    iota16 = lax.iota(jnp.int32, 16)
    bvecs = [iota16 + (g * 16) for g in range(_LANES // 16)]

    def compute_gidx(t, k):
        for g in range(_LANES // 16):
            tok = idx_all[t, pl.ds(g * 16, 16)]
            gidx[k][pl.ds(g * 16, 16)] = lax.shift_right_logical(tok, 1)

    def gdesc(k):
        return pltpu.make_async_copy(tab2_hbm.at[gidx[k]], emb[k], gsem)

    def odesc(t, k):
        return pltpu.make_async_copy(
            outb[k], out_hbm.at[t, :, pl.ds(b0, _LANES)], osem)

    def transpose_add(t, k):
        pv = tuple((idx_all[t, pl.ds(g * 16, 16)] & 1) * NEMBED
                   for g in range(_LANES // 16))

        zero16 = iota16 * 0
        t_splat = zero16 + t

        def c_body(c, pv):
            ps = plsc.load_gather(pos_v, [t_splat, zero16 + c])
            for g in range(_LANES // 16):
                val = plsc.load_gather(emb[k], [bvecs[g], pv[g] + c])
                outb[k][c, pl.ds(g * 16, 16)] = val + ps
            return pv

        lax.fori_loop(0, NEMBED, c_body, pv, unroll=False)

    for t in range(2):
        compute_gidx(t, t)
        gdesc(t).start()

    def loop_body(t2, c):
        for par in range(2):
            t = t2 * 2 + par
            k = par
            gdesc(k).wait()

            @pl.when(t >= 2)
            def _():
                odesc(t - 2, k).wait()

            transpose_add(t, k)
            odesc(t, k).start()

            @pl.when(t + 2 < NTOKEN)
            def _():
                compute_gidx(t + 2, k)
                gdesc(k).start()
        return c

    lax.fori_loop(0, NTOKEN // 2, loop_body, 0, unroll=False)

    odesc(NTOKEN - 2, 0).wait()
    odesc(NTOKEN - 1, 1).wait()


@jax.jit
def kernel(tokens, token_table, pos_embed):
    mesh = plsc.VectorSubcoreMesh(core_axis_name="c", subcore_axis_name="s")
    params = pltpu.CompilerParams(
        use_tc_tiling_on_sc=True, needs_layout_passes=False)

    tab2 = pl.kernel(
        _transpose_kernel,
        out_type=jax.ShapeDtypeStruct((_T2ROWS, 128), jnp.float32),
        mesh=mesh,
        scratch_types=[
            [pltpu.VMEM((NEMBED, 128), jnp.float32) for _ in range(2)],
            [pltpu.VMEM((NEMBED, 128), jnp.float32) for _ in range(2)],
            pltpu.VMEM((_TAIL, NEMBED), jnp.float32),
            pltpu.SemaphoreType.DMA,
            pltpu.SemaphoreType.DMA,
        ],
        compiler_params=params,
    )(token_table.T, token_table[_NTB * 128:, :])

    tokT = tokens.astype(jnp.int32).T
    out_phys = pl.kernel(
        _gather_kernel,
        out_type=jax.ShapeDtypeStruct((NTOKEN, NEMBED, BATCH), jnp.float32),
        mesh=mesh,
        scratch_types=[
            pltpu.VMEM((NTOKEN, _LANES), jnp.int32),
            pltpu.VMEM((NTOKEN, NEMBED), jnp.float32),
            [pltpu.VMEM((_LANES,), jnp.int32) for _ in range(2)],
            [pltpu.VMEM((_LANES, 128), jnp.float32) for _ in range(2)],
            [pltpu.VMEM((NEMBED, _LANES), jnp.float32) for _ in range(2)],
            pltpu.SemaphoreType.DMA,
            pltpu.SemaphoreType.DMA,
        ],
        compiler_params=params,
    )(tokT, tab2, pos_embed)
    return out_phys.transpose(2, 0, 1)


# speedup vs baseline: 1.1869x; 1.1869x over previous
import jax
import jax.numpy as jnp
from jax import lax
from jax.experimental import pallas as pl
from jax.experimental.pallas import tpu as pltpu
from jax.experimental.pallas import tpu_sc as plsc

NVOCAB = 1000000
NEMBED = 64
NTOKEN = 200
BATCH = 4096

_NUM_WORKERS = 32
_LANES = BATCH // _NUM_WORKERS
_T2ROWS = NVOCAB // 2
_NTB = NVOCAB // 128
_TB_PER_W = _NTB // _NUM_WORKERS
_TB_REM = _NTB % _NUM_WORKERS
_TAIL = NVOCAB - _NTB * 128


def _transpose_kernel(tabT_hbm, tail_hbm, out_hbm, src, dst, tail_v,
                      gsem, osem):
    nc = 2
    wid = lax.axis_index("s") * nc + lax.axis_index("c")

    iota16 = lax.iota(jnp.int32, 16)
    cvecs = [iota16 + (g * 16) for g in range(4)]

    def gdesc(tb, k):
        return pltpu.make_async_copy(
            tabT_hbm.at[:, pl.ds(tb * 128, 128)], src[k], gsem)

    def odesc(tb, k):
        return pltpu.make_async_copy(
            dst[k], out_hbm.at[pl.ds(tb * 64, 64)], osem)

    def transpose(k):
        zero16 = iota16 * 0
        for h in range(2):

            def r_body(r, c):
                col = zero16 + (2 * r + h)
                vals = [plsc.load_gather(src[k], [cvecs[g], col])
                        for g in range(4)]
                for g in range(4):
                    dst[k][r, pl.ds(h * NEMBED + g * 16, 16)] = vals[g]
                return c

            lax.fori_loop(0, NEMBED, r_body, 0, unroll=4)

    n_iters = _TB_PER_W + 1

    def tb_of(i):
        return wid + i * _NUM_WORKERS

    def start(i, k):
        tb = tb_of(i)

        @pl.when(tb < _NTB)
        def _():
            gdesc(tb, k).start()

    def finish(i, k):
        tb = tb_of(i)

        @pl.when(tb < _NTB)
        def _():
            gdesc(tb, k).wait()

            @pl.when(i >= 2)
            def _():
                odesc(tb_of(i - 2), k).wait()

            transpose(k)
            odesc(tb, k).start()

    start(0, 0)
    start(1, 1)

    def body2(i2, c):
        for par in range(2):
            i = i2 * 2 + par
            finish(i, par)
            start(i + 2, par)
        return c

    n_pairs = (n_iters + 1) // 2
    lax.fori_loop(0, n_pairs, body2, 0, unroll=False)

    for i in (2 * n_pairs - 4, 2 * n_pairs - 3, 2 * n_pairs - 2):
        tb = tb_of(i)

        @pl.when((tb < _NTB) & (tb + 2 * _NUM_WORKERS >= _NTB))
        def _():
            odesc(tb, i % 2).wait()

    @pl.when(wid == _NTB % _NUM_WORKERS)
    def _():
        pltpu.sync_copy(tail_hbm, tail_v)

        def r_body(r, c):
            for h in range(2):
                col = 2 * r + h
                for g in range(4):
                    dst[0][r, pl.ds(h * NEMBED + g * 16, 16)] = (
                        tail_v[col, pl.ds(g * 16, 16)])
            return c

        lax.fori_loop(0, _TAIL // 2, r_body, 0, unroll=False)
        pltpu.sync_copy(dst[0].at[pl.ds(0, _TAIL // 2)],
                        out_hbm.at[pl.ds(_NTB * 64, _TAIL // 2)])


def _gather_kernel(tokT_hbm, tab2_hbm, posT_hbm, out_hbm,
                   idx_all, posT_v, gidx, emb, outb, gsem, osem):
    nc = 2
    wid = lax.axis_index("s") * nc + lax.axis_index("c")
    b0 = wid * _LANES

    pltpu.sync_copy(tokT_hbm.at[:, pl.ds(b0, _LANES)], idx_all)
    pltpu.sync_copy(posT_hbm, posT_v)

    iota16 = lax.iota(jnp.int32, 16)
    bvecs = [iota16 + (g * 16) for g in range(_LANES // 16)]

    def compute_gidx(t, k):
        for g in range(_LANES // 16):
            tok = idx_all[t, pl.ds(g * 16, 16)]
            gidx[k][pl.ds(g * 16, 16)] = lax.shift_right_logical(tok, 1)

    def gdesc(k):
        return pltpu.make_async_copy(tab2_hbm.at[gidx[k]], emb[k], gsem)

    def odesc(t, ko):
        return pltpu.make_async_copy(
            outb[ko], out_hbm.at[t, :, pl.ds(b0, _LANES)], osem)

    def transpose_add(t, k, ko):
        for gp in range(_LANES // 64):
            gs = [gp * 4 + j for j in range(4)]
            pvs = tuple((idx_all[t, pl.ds(g * 16, 16)] & 1) * NEMBED
                        for g in gs)

            def c_body(c, pv):
                vals = [plsc.load_gather(emb[k], [bvecs[g], pv[j] + c])
                        for j, g in enumerate(gs)]
                for j, g in enumerate(gs):
                    outb[ko][c, pl.ds(g * 16, 16)] = vals[j]
                return pv

            lax.fori_loop(0, NEMBED, c_body, pvs, unroll=4)

        zero16 = iota16 * 0
        t_splat = zero16 + t

        def p_body(c, carry):
            ps = plsc.load_gather(posT_v, [zero16 + c, t_splat])
            for g in range(_LANES // 16):
                plsc.addupdate(outb[ko].at[c, pl.ds(g * 16, 16)], ps)
            return carry

        lax.fori_loop(0, NEMBED, p_body, 0, unroll=8)

    for t in range(3):
        compute_gidx(t, t)
        gdesc(t).start()

    def loop_body(t3, c):
        for par in range(3):
            t = t3 * 3 + par
            k = par

            @pl.when(t < NTOKEN)
            def _():
                gdesc(k).wait()

                @pl.when(t >= 3)
                def _():
                    odesc(t - 3, k).wait()

                transpose_add(t, k, k)
                odesc(t, k).start()

                @pl.when(t + 3 < NTOKEN)
                def _():
                    compute_gidx(t + 3, k)
                    gdesc(k).start()
        return c

    lax.fori_loop(0, (NTOKEN + 2) // 3, loop_body, 0, unroll=False)

    for tt in (NTOKEN - 3, NTOKEN - 2, NTOKEN - 1):
        odesc(tt, tt % 3).wait()


@jax.jit
def kernel(tokens, token_table, pos_embed):
    mesh = plsc.VectorSubcoreMesh(core_axis_name="c", subcore_axis_name="s")
    params = pltpu.CompilerParams(
        use_tc_tiling_on_sc=True, needs_layout_passes=False)

    tab2 = pl.kernel(
        _transpose_kernel,
        out_type=jax.ShapeDtypeStruct((_T2ROWS, 128), jnp.float32),
        mesh=mesh,
        scratch_types=[
            [pltpu.VMEM((NEMBED, 128), jnp.float32) for _ in range(2)],
            [pltpu.VMEM((NEMBED, 128), jnp.float32) for _ in range(2)],
            pltpu.VMEM((_TAIL, NEMBED), jnp.float32),
            pltpu.SemaphoreType.DMA,
            pltpu.SemaphoreType.DMA,
        ],
        compiler_params=params,
    )(token_table.T, token_table[_NTB * 128:, :])

    tokT = tokens.astype(jnp.int32).T
    out_phys = pl.kernel(
        _gather_kernel,
        out_type=jax.ShapeDtypeStruct((NTOKEN, NEMBED, BATCH), jnp.float32),
        mesh=mesh,
        scratch_types=[
            pltpu.VMEM((NTOKEN, _LANES), jnp.int32),
            pltpu.VMEM((NEMBED, NTOKEN), jnp.float32),
            [pltpu.VMEM((_LANES,), jnp.int32) for _ in range(3)],
            [pltpu.VMEM((_LANES, 128), jnp.float32) for _ in range(3)],
            [pltpu.VMEM((NEMBED, _LANES), jnp.float32) for _ in range(3)],
            pltpu.SemaphoreType.DMA,
            pltpu.SemaphoreType.DMA,
        ],
        compiler_params=params,
    )(tokT, tab2, pos_embed.T)
    return out_phys.transpose(2, 0, 1)
